# baseline (speedup 1.0000x reference)
import functools

import jax
import jax.numpy as jnp
from jax import lax
from jax.experimental import pallas as pl
from jax.experimental.pallas import tpu as pltpu
from jax.experimental.pallas import tpu_sc as plsc

_N_FIELDS = 26
_VOCAB = 64
_BATCH = 16384
_LANES = 16
_NC = 1
_NS = 16
_NW = _NC * _NS
_BPW = _BATCH // _NW
_GROUPS = _BPW // _LANES


def _body(x_hbm, tab_hbm, out_hbm, x_v, tab_v, out_v, sem):
    wid = lax.axis_index("s") * _NC + lax.axis_index("c")
    base = wid * _BPW

    cp = pltpu.async_copy(
        x_hbm.at[pl.ds(base * _N_FIELDS, _BPW * _N_FIELDS)], x_v, sem)
    pltpu.sync_copy(tab_hbm, tab_v)
    cp.wait()

    row_off = lax.iota(jnp.int32, _LANES) * _N_FIELDS

    def group(g, carry):
        x_base = row_off + g * (_LANES * _N_FIELDS)
        xi = plsc.load_gather(x_v, [x_base])
        acc = plsc.load_gather(tab_v, [xi])
        for i in range(1, _N_FIELDS):
            xi = plsc.load_gather(x_v, [x_base + i])
            acc = acc + plsc.load_gather(tab_v, [xi + i * _VOCAB])
        out_v[pl.ds(g * _LANES, _LANES)] = acc
        return carry

    lax.fori_loop(0, _GROUPS, group, 0)
    pltpu.sync_copy(out_v, out_hbm.at[pl.ds(base, _BPW)])


@jax.jit
def kernel(X, tables, free_term):
    mesh = plsc.VectorSubcoreMesh(core_axis_name="c", subcore_axis_name="s", num_cores=1)
    run = functools.partial(
        pl.kernel,
        out_type=jax.ShapeDtypeStruct((_BATCH,), jnp.float32),
        mesh=mesh,
        scratch_types=[
            pltpu.VMEM((_BPW * _N_FIELDS,), jnp.int32),
            pltpu.VMEM((_N_FIELDS * _VOCAB,), jnp.float32),
            pltpu.VMEM((_BPW,), jnp.float32),
            pltpu.SemaphoreType.DMA,
        ],
        compiler_params=pltpu.CompilerParams(
            needs_layout_passes=False, skip_device_barrier=True),
    )(_body)
    tab = tables.astype(jnp.float32).at[0].add(free_term.astype(jnp.float32))
    return run(X.reshape(-1), tab.reshape(-1))


# speedup vs baseline: 1.0014x; 1.0014x over previous
import functools

import jax
import jax.numpy as jnp
from jax import lax
from jax.experimental import pallas as pl
from jax.experimental.pallas import tpu as pltpu
from jax.experimental.pallas import tpu_sc as plsc

_N_FIELDS = 26
_VOCAB = 64
_BATCH = 16384
_LANES = 16
_NC = 1
_NS = 16
_NW = _NC * _NS
_BPW = _BATCH // _NW
_GROUPS = _BPW // _LANES


def _body(x_hbm, tab_hbm, out_hbm, x_v, tab_v, out_v, sem):
    wid = lax.axis_index("s") * _NC + lax.axis_index("c")
    base = wid * _BPW

    cp = pltpu.async_copy(
        x_hbm.at[pl.ds(base * _N_FIELDS, _BPW * _N_FIELDS)], x_v, sem)
    pltpu.sync_copy(tab_hbm, tab_v)
    cp.wait()

    row_off = lax.iota(jnp.int32, _LANES) * _N_FIELDS

    def group(g, carry):
        x_base = row_off + g * (_LANES * _N_FIELDS)
        xi = plsc.load_gather(x_v, [x_base])
        acc = plsc.load_gather(tab_v, [xi])
        for i in range(1, _N_FIELDS):
            xi = plsc.load_gather(x_v, [x_base + i])
            acc = acc + plsc.load_gather(tab_v, [xi + i * _VOCAB])
        out_v[pl.ds(g * _LANES, _LANES)] = acc
        return carry

    lax.fori_loop(0, _GROUPS, group, 0)
    pltpu.sync_copy(out_v, out_hbm.at[pl.ds(base, _BPW)])


@jax.jit
def kernel(X, tables, free_term):
    mesh = plsc.VectorSubcoreMesh(core_axis_name="c", subcore_axis_name="s", num_cores=1)
    run = functools.partial(
        pl.kernel,
        out_type=jax.ShapeDtypeStruct((_BATCH,), jnp.float32),
        mesh=mesh,
        scratch_types=[
            pltpu.VMEM((_BPW * _N_FIELDS,), jnp.int32),
            pltpu.VMEM((_N_FIELDS * _VOCAB,), jnp.float32),
            pltpu.VMEM((_BPW,), jnp.float32),
            pltpu.SemaphoreType.DMA,
        ],
        compiler_params=pltpu.CompilerParams(needs_layout_passes=False),
    )(_body)
    tab = tables.astype(jnp.float32).at[0].add(free_term.astype(jnp.float32))
    return run(X.reshape(-1), tab.reshape(-1))
